# Initial kernel scaffold; baseline (speedup 1.0000x reference)
import jax
import jax.numpy as jnp
from jax.experimental import pallas as pl


def kernel(subgraph, feat, norm, send_map, recv_map, rank, size, W1, b1, W2, b2, W3, b3, W4, b4, W5, b5, W6, b6):
    raise NotImplementedError("write your pallas kernel here")


# speedup vs baseline: 66.6700x; 66.6700x over previous
import functools

import jax
import jax.numpy as jnp
from jax import lax
from jax.experimental import pallas as pl
from jax.experimental.pallas import tpu as pltpu
from jax.experimental.pallas import tpu_sc as plsc

N_NODES = 100000
N_EDGES = 6400000
NPAD = 102400
NTILES = 16
NCORES = 2
NCH = NPAD // NTILES
ECH = N_EDGES // (NCORES * NTILES)
CHUNK = 2000
NCHUNKS = ECH // CHUNK


def _make_agg(F):
    scratch = (
        [pltpu.VMEM_SHARED((NPAD,), jnp.float32) for _ in range(F)]
        + [pltpu.VMEM_SHARED((NPAD,), jnp.float32) for _ in range(F)]
        + [
            pltpu.VMEM((CHUNK,), jnp.int32),
            pltpu.VMEM((CHUNK,), jnp.int32),
            pltpu.VMEM((CHUNK,), jnp.float32),
            pltpu.VMEM((NCH,), jnp.float32),
        ]
    )

    @functools.partial(
        pl.kernel,
        mesh=plsc.VectorSubcoreMesh(core_axis_name="c", subcore_axis_name="s"),
        out_type=jax.ShapeDtypeStruct((NCORES * F * NPAD,), jnp.float32),
        scratch_types=scratch,
    )
    def agg_kernel(src, dst, x, out, *refs):
        xs = refs[:F]
        ag = refs[F:2 * F]
        srcb, dstb, valb, nodeb = refs[2 * F:]
        c = lax.axis_index("c")
        s = lax.axis_index("s")
        nlo = s * NCH
        for f in range(F):
            pltpu.sync_copy(x.at[pl.ds(f * NPAD + nlo, NCH)], nodeb)
            pltpu.sync_copy(nodeb, xs[f].at[pl.ds(nlo, NCH)])
        zero = jnp.zeros((16,), jnp.float32)

        def zbody(i, carry):
            nodeb[pl.ds(i * 16, 16)] = zero
            return carry

        lax.fori_loop(0, NCH // 16, zbody, 0)
        for f in range(F):
            pltpu.sync_copy(nodeb, ag[f].at[pl.ds(nlo, NCH)])
        plsc.subcore_barrier()

        ebase = (c * NTILES + s) * ECH

        def echunk(i, carry):
            off = ebase + i * CHUNK
            pltpu.sync_copy(src.at[pl.ds(off, CHUNK)], srcb)
            pltpu.sync_copy(dst.at[pl.ds(off, CHUNK)], dstb)
            for f in range(F):
                pltpu.sync_copy(xs[f].at[srcb], valb)
                pltpu.sync_copy(valb, ag[f].at[dstb], add=True)
            return carry

        lax.fori_loop(0, NCHUNKS, echunk, 0)
        plsc.subcore_barrier()
        for f in range(F):
            pltpu.sync_copy(ag[f].at[pl.ds(nlo, NCH)], nodeb)
            pltpu.sync_copy(nodeb, out.at[pl.ds((c * F + f) * NPAD + nlo, NCH)])

    return agg_kernel


_agg1 = _make_agg(1)
_agg3 = _make_agg(3)


def _pad_planes(x):
    F = x.shape[1]
    return jnp.zeros((F, NPAD), x.dtype).at[:, :N_NODES].set(x.T).reshape(-1)


def _aggregate(agg_fn, src, dst, x):
    F = x.shape[1]
    parts = agg_fn(src, dst, _pad_planes(x)).reshape(NCORES, F, NPAD)
    s = parts[0] + parts[1]
    return s[:, :N_NODES].T


def kernel(subgraph, feat, norm, send_map, recv_map, rank, size,
           W1, b1, W2, b2, W3, b3, W4, b4, W5, b5, W6, b6):
    src = subgraph[0]
    dst = subgraph[1]
    a = _aggregate(_agg1, src, dst, feat * norm)
    h = jax.nn.relu((norm * a) @ W1 + b1)
    for W, b in ((W2, b2), (W3, b3), (W4, b4), (W5, b5)):
        a = _aggregate(_agg3, src, dst, norm * h)
        h = jax.nn.relu((norm * a) @ W + b)
    y = (norm * h) @ W6
    a = _aggregate(_agg1, src, dst, y)
    return jax.nn.relu(norm * a + b6)


# speedup vs baseline: 83.7715x; 1.2565x over previous
import functools

import jax
import jax.numpy as jnp
from jax import lax
from jax.experimental import pallas as pl
from jax.experimental.pallas import tpu as pltpu
from jax.experimental.pallas import tpu_sc as plsc

N_NODES = 100000
N_EDGES = 6400000
NPAD = 102400
NTILES = 16
NCORES = 2
NCH = NPAD // NTILES
ECH = N_EDGES // (NCORES * NTILES)
CHUNK = 4000
NCHUNKS = ECH // CHUNK


def _make_agg(F):
    scratch = (
        [pltpu.VMEM_SHARED((NPAD,), jnp.float32) for _ in range(F)]
        + [pltpu.VMEM_SHARED((NPAD,), jnp.float32) for _ in range(F)]
        + [
            pltpu.VMEM((CHUNK,), jnp.int32),
            pltpu.VMEM((CHUNK,), jnp.int32),
            pltpu.VMEM((CHUNK,), jnp.int32),
            pltpu.VMEM((CHUNK,), jnp.int32),
            pltpu.VMEM((CHUNK,), jnp.float32),
            pltpu.VMEM((NCH,), jnp.float32),
            pltpu.SemaphoreType.DMA,
            pltpu.SemaphoreType.DMA,
            pltpu.SemaphoreType.DMA,
            pltpu.SemaphoreType.DMA,
        ]
    )

    @functools.partial(
        pl.kernel,
        mesh=plsc.VectorSubcoreMesh(core_axis_name="c", subcore_axis_name="s"),
        out_type=jax.ShapeDtypeStruct((NCORES * F * NPAD,), jnp.float32),
        scratch_types=scratch,
    )
    def agg_kernel(src, dst, x, out, *refs):
        xs = refs[:F]
        ag = refs[F:2 * F]
        (srcb0, srcb1, dstb0, dstb1, valb, nodeb,
         sem_s0, sem_s1, sem_d0, sem_d1) = refs[2 * F:]
        srcbufs, dstbufs = (srcb0, srcb1), (dstb0, dstb1)
        sems_s, sems_d = (sem_s0, sem_s1), (sem_d0, sem_d1)
        c = lax.axis_index("c")
        s = lax.axis_index("s")
        nlo = s * NCH
        for f in range(F):
            pltpu.sync_copy(x.at[pl.ds(f * NPAD + nlo, NCH)], nodeb)
            pltpu.sync_copy(nodeb, xs[f].at[pl.ds(nlo, NCH)])
        zero = jnp.zeros((16,), jnp.float32)

        def zbody(i, carry):
            nodeb[pl.ds(i * 16, 16)] = zero
            return carry

        lax.fori_loop(0, NCH // 16, zbody, 0)
        for f in range(F):
            pltpu.sync_copy(nodeb, ag[f].at[pl.ds(nlo, NCH)])
        plsc.subcore_barrier()

        ebase = (c * NTILES + s) * ECH

        def load_copies(g, b):
            off = ebase + g * CHUNK
            return (
                pltpu.make_async_copy(src.at[pl.ds(off, CHUNK)], srcbufs[b],
                                      sems_s[b]),
                pltpu.make_async_copy(dst.at[pl.ds(off, CHUNK)], dstbufs[b],
                                      sems_d[b]),
            )

        def start_loads(g, b):
            for cp in load_copies(g, b):
                cp.start()

        def wait_loads(g, b):
            for cp in load_copies(g, b):
                cp.wait()

        def process(b):
            for f in range(F):
                pltpu.sync_copy(xs[f].at[srcbufs[b]], valb)
                pltpu.sync_copy(valb, ag[f].at[dstbufs[b]], add=True)

        start_loads(0, 0)

        def epair(p, carry):
            g0 = p * 2
            start_loads(g0 + 1, 1)
            wait_loads(g0, 0)
            process(0)

            @pl.when(p < NCHUNKS // 2 - 1)
            def _():
                start_loads(g0 + 2, 0)

            wait_loads(g0 + 1, 1)
            process(1)
            return carry

        lax.fori_loop(0, NCHUNKS // 2, epair, 0)
        plsc.subcore_barrier()
        for f in range(F):
            pltpu.sync_copy(ag[f].at[pl.ds(nlo, NCH)], nodeb)
            pltpu.sync_copy(nodeb, out.at[pl.ds((c * F + f) * NPAD + nlo, NCH)])

    return agg_kernel


_agg1 = _make_agg(1)
_agg3 = _make_agg(3)


def _pad_planes(x):
    F = x.shape[1]
    return jnp.zeros((F, NPAD), x.dtype).at[:, :N_NODES].set(x.T).reshape(-1)


def _aggregate(agg_fn, src, dst, x):
    F = x.shape[1]
    parts = agg_fn(src, dst, _pad_planes(x)).reshape(NCORES, F, NPAD)
    s = parts[0] + parts[1]
    return s[:, :N_NODES].T


def kernel(subgraph, feat, norm, send_map, recv_map, rank, size,
           W1, b1, W2, b2, W3, b3, W4, b4, W5, b5, W6, b6):
    src = subgraph[0]
    dst = subgraph[1]
    a = _aggregate(_agg1, src, dst, feat * norm)
    h = jax.nn.relu((norm * a) @ W1 + b1)
    for W, b in ((W2, b2), (W3, b3), (W4, b4), (W5, b5)):
        a = _aggregate(_agg3, src, dst, norm * h)
        h = jax.nn.relu((norm * a) @ W + b)
    y = (norm * h) @ W6
    a = _aggregate(_agg1, src, dst, y)
    return jax.nn.relu(norm * a + b6)


# speedup vs baseline: 85.5704x; 1.0215x over previous
import functools

import jax
import jax.numpy as jnp
from jax import lax
from jax.experimental import pallas as pl
from jax.experimental.pallas import tpu as pltpu
from jax.experimental.pallas import tpu_sc as plsc

N_NODES = 100000
N_EDGES = 6400000
NPAD = 102400
NTILES = 16
NCORES = 2
NCH = NPAD // NTILES
ECH = N_EDGES // (NCORES * NTILES)
CHUNK = 4000
NCHUNKS = ECH // CHUNK


def _make_agg(F):
    scratch = (
        [pltpu.VMEM_SHARED((NPAD,), jnp.float32) for _ in range(F)]
        + [pltpu.VMEM_SHARED((NPAD,), jnp.float32) for _ in range(F)]
        + [
            pltpu.VMEM((CHUNK,), jnp.int32),
            pltpu.VMEM((CHUNK,), jnp.int32),
            pltpu.VMEM((CHUNK,), jnp.int32),
            pltpu.VMEM((CHUNK,), jnp.int32),
            pltpu.VMEM((CHUNK,), jnp.float32),
            pltpu.VMEM((CHUNK,), jnp.float32),
            pltpu.VMEM((NCH,), jnp.float32),
            pltpu.SemaphoreType.DMA,
            pltpu.SemaphoreType.DMA,
            pltpu.SemaphoreType.DMA,
            pltpu.SemaphoreType.DMA,
            pltpu.SemaphoreType.DMA,
            pltpu.SemaphoreType.DMA,
        ]
    )

    @functools.partial(
        pl.kernel,
        mesh=plsc.VectorSubcoreMesh(core_axis_name="c", subcore_axis_name="s"),
        out_type=jax.ShapeDtypeStruct((NCORES * F * NPAD,), jnp.float32),
        scratch_types=scratch,
    )
    def agg_kernel(src, dst, x, out, *refs):
        xs = refs[:F]
        ag = refs[F:2 * F]
        (srcb0, srcb1, dstb0, dstb1, valb0, valb1, nodeb,
         sem_s0, sem_s1, sem_d0, sem_d1, sem_g0, sem_g1) = refs[2 * F:]
        srcbufs, dstbufs = (srcb0, srcb1), (dstb0, dstb1)
        valbufs, sems_g = (valb0, valb1), (sem_g0, sem_g1)
        sems_s, sems_d = (sem_s0, sem_s1), (sem_d0, sem_d1)
        c = lax.axis_index("c")
        s = lax.axis_index("s")
        nlo = s * NCH
        for f in range(F):
            pltpu.sync_copy(x.at[pl.ds(f * NPAD + nlo, NCH)], nodeb)
            pltpu.sync_copy(nodeb, xs[f].at[pl.ds(nlo, NCH)])
        zero = jnp.zeros((16,), jnp.float32)

        def zbody(i, carry):
            nodeb[pl.ds(i * 16, 16)] = zero
            return carry

        lax.fori_loop(0, NCH // 16, zbody, 0)
        for f in range(F):
            pltpu.sync_copy(nodeb, ag[f].at[pl.ds(nlo, NCH)])
        plsc.subcore_barrier()

        ebase = (c * NTILES + s) * ECH

        def load_copies(g, b):
            off = ebase + g * CHUNK
            return (
                pltpu.make_async_copy(src.at[pl.ds(off, CHUNK)], srcbufs[b],
                                      sems_s[b]),
                pltpu.make_async_copy(dst.at[pl.ds(off, CHUNK)], dstbufs[b],
                                      sems_d[b]),
            )

        def start_loads(g, b):
            for cp in load_copies(g, b):
                cp.start()

        def wait_loads(g, b):
            for cp in load_copies(g, b):
                cp.wait()

        def process(b):
            def gather(f):
                v = f % 2
                return pltpu.make_async_copy(xs[f].at[srcbufs[b]],
                                             valbufs[v], sems_g[v])

            gather(0).start()
            for f in range(F):
                if f + 1 < F:
                    gather(f + 1).start()
                gather(f).wait()
                pltpu.sync_copy(valbufs[f % 2], ag[f].at[dstbufs[b]], add=True)

        start_loads(0, 0)

        def epair(p, carry):
            g0 = p * 2
            start_loads(g0 + 1, 1)
            wait_loads(g0, 0)
            process(0)

            @pl.when(p < NCHUNKS // 2 - 1)
            def _():
                start_loads(g0 + 2, 0)

            wait_loads(g0 + 1, 1)
            process(1)
            return carry

        lax.fori_loop(0, NCHUNKS // 2, epair, 0)
        plsc.subcore_barrier()
        for f in range(F):
            pltpu.sync_copy(ag[f].at[pl.ds(nlo, NCH)], nodeb)
            pltpu.sync_copy(nodeb, out.at[pl.ds((c * F + f) * NPAD + nlo, NCH)])

    return agg_kernel


_agg1 = _make_agg(1)
_agg3 = _make_agg(3)


def _pad_planes(x):
    F = x.shape[1]
    return jnp.zeros((F, NPAD), x.dtype).at[:, :N_NODES].set(x.T).reshape(-1)


def _aggregate(agg_fn, src, dst, x):
    F = x.shape[1]
    parts = agg_fn(src, dst, _pad_planes(x)).reshape(NCORES, F, NPAD)
    s = parts[0] + parts[1]
    return s[:, :N_NODES].T


def kernel(subgraph, feat, norm, send_map, recv_map, rank, size,
           W1, b1, W2, b2, W3, b3, W4, b4, W5, b5, W6, b6):
    src = subgraph[0]
    dst = subgraph[1]
    a = _aggregate(_agg1, src, dst, feat * norm)
    h = jax.nn.relu((norm * a) @ W1 + b1)
    for W, b in ((W2, b2), (W3, b3), (W4, b4), (W5, b5)):
        a = _aggregate(_agg3, src, dst, norm * h)
        h = jax.nn.relu((norm * a) @ W + b)
    y = (norm * h) @ W6
    a = _aggregate(_agg1, src, dst, y)
    return jax.nn.relu(norm * a + b6)


# speedup vs baseline: 86.0804x; 1.0060x over previous
import functools

import jax
import jax.numpy as jnp
from jax import lax
from jax.experimental import pallas as pl
from jax.experimental.pallas import tpu as pltpu
from jax.experimental.pallas import tpu_sc as plsc

N_NODES = 100000
N_EDGES = 6400000
NPAD = 102400
NTILES = 16
NCORES = 2
NCH = NPAD // NTILES
ECH = N_EDGES // (NCORES * NTILES)
CHUNK = 4000
NCHUNKS = ECH // CHUNK


def _make_agg(F):
    scratch = (
        [pltpu.VMEM_SHARED((NPAD,), jnp.float32) for _ in range(F)]
        + [pltpu.VMEM_SHARED((NPAD,), jnp.float32) for _ in range(F)]
        + [
            pltpu.VMEM((CHUNK,), jnp.int32),
            pltpu.VMEM((CHUNK,), jnp.int32),
            pltpu.VMEM((CHUNK,), jnp.int32),
            pltpu.VMEM((CHUNK,), jnp.int32),
            pltpu.VMEM((CHUNK,), jnp.float32),
            pltpu.VMEM((CHUNK,), jnp.float32),
            pltpu.VMEM((NCH,), jnp.float32),
            pltpu.VMEM((NCH,), jnp.float32),
            pltpu.VMEM((NCH,), jnp.float32),
            pltpu.VMEM((NCH,), jnp.float32),
            pltpu.SemaphoreType.DMA,
            pltpu.SemaphoreType.DMA,
            pltpu.SemaphoreType.DMA,
            pltpu.SemaphoreType.DMA,
            pltpu.SemaphoreType.DMA,
            pltpu.SemaphoreType.DMA,
            pltpu.SemaphoreType.DMA,
            pltpu.SemaphoreType.DMA,
            pltpu.SemaphoreType.DMA,
            pltpu.SemaphoreType.DMA,
            pltpu.SemaphoreType.DMA,
        ]
    )

    @functools.partial(
        pl.kernel,
        mesh=plsc.VectorSubcoreMesh(core_axis_name="c", subcore_axis_name="s"),
        out_type=jax.ShapeDtypeStruct((NCORES * F * NPAD,), jnp.float32),
        scratch_types=scratch,
    )
    def agg_kernel(src, dst, x, out, *refs):
        xs = refs[:F]
        ag = refs[F:2 * F]
        (srcb0, srcb1, dstb0, dstb1, valb0, valb1, pb0, pb1, pb2, zbuf,
         sem_s0, sem_s1, sem_d0, sem_d1, sem_g0, sem_g1,
         sem_p0, sem_p1, sem_p2, sem_fwd, sem_z) = refs[2 * F:]
        srcbufs, dstbufs = (srcb0, srcb1), (dstb0, dstb1)
        valbufs, sems_g = (valb0, valb1), (sem_g0, sem_g1)
        sems_s, sems_d = (sem_s0, sem_s1), (sem_d0, sem_d1)
        pbufs, sems_p = (pb0, pb1, pb2), (sem_p0, sem_p1, sem_p2)
        c = lax.axis_index("c")
        s = lax.axis_index("s")
        nlo = s * NCH
        nsl = pl.ds(nlo, NCH)

        fetches = [pltpu.make_async_copy(x.at[pl.ds(f * NPAD + nlo, NCH)],
                                         pbufs[f], sems_p[f]) for f in range(F)]
        for cp in fetches:
            cp.start()
        zero = jnp.zeros((16,), jnp.float32)

        def zbody(i, carry):
            zbuf[pl.ds(i * 16, 16)] = zero
            return carry

        lax.fori_loop(0, NCH // 16, zbody, 0)
        zcps = [pltpu.make_async_copy(zbuf, ag[f].at[nsl], sem_z)
                for f in range(F)]
        for cp in zcps:
            cp.start()
        fwds = [pltpu.make_async_copy(pbufs[f], xs[f].at[nsl], sem_fwd)
                for f in range(F)]
        for f in range(F):
            fetches[f].wait()
            fwds[f].start()
        for f in range(F):
            zcps[f].wait()
            fwds[f].wait()
        plsc.subcore_barrier()

        ebase = (c * NTILES + s) * ECH

        def load_copies(g, b):
            off = ebase + g * CHUNK
            return (
                pltpu.make_async_copy(src.at[pl.ds(off, CHUNK)], srcbufs[b],
                                      sems_s[b]),
                pltpu.make_async_copy(dst.at[pl.ds(off, CHUNK)], dstbufs[b],
                                      sems_d[b]),
            )

        def start_loads(g, b):
            for cp in load_copies(g, b):
                cp.start()

        def wait_loads(g, b):
            for cp in load_copies(g, b):
                cp.wait()

        def process(b):
            def gather(f):
                v = f % 2
                return pltpu.make_async_copy(xs[f].at[srcbufs[b]],
                                             valbufs[v], sems_g[v])

            gather(0).start()
            for f in range(F):
                if f + 1 < F:
                    gather(f + 1).start()
                gather(f).wait()
                pltpu.sync_copy(valbufs[f % 2], ag[f].at[dstbufs[b]], add=True)

        start_loads(0, 0)

        def epair(p, carry):
            g0 = p * 2
            start_loads(g0 + 1, 1)
            wait_loads(g0, 0)
            process(0)

            @pl.when(p < NCHUNKS // 2 - 1)
            def _():
                start_loads(g0 + 2, 0)

            wait_loads(g0 + 1, 1)
            process(1)
            return carry

        lax.fori_loop(0, NCHUNKS // 2, epair, 0)
        plsc.subcore_barrier()
        ocps = [pltpu.make_async_copy(ag[f].at[nsl], pbufs[f], sems_p[f])
                for f in range(F)]
        for cp in ocps:
            cp.start()
        wbs = [pltpu.make_async_copy(
                   pbufs[f], out.at[pl.ds((c * F + f) * NPAD + nlo, NCH)],
                   sem_fwd) for f in range(F)]
        for f in range(F):
            ocps[f].wait()
            wbs[f].start()
        for f in range(F):
            wbs[f].wait()

    return agg_kernel


_agg1 = _make_agg(1)
_agg3 = _make_agg(3)


def _pad_planes(x):
    F = x.shape[1]
    return jnp.zeros((F, NPAD), x.dtype).at[:, :N_NODES].set(x.T).reshape(-1)


def _aggregate(agg_fn, src, dst, x):
    F = x.shape[1]
    parts = agg_fn(src, dst, _pad_planes(x)).reshape(NCORES, F, NPAD)
    s = parts[0] + parts[1]
    return s[:, :N_NODES].T


def kernel(subgraph, feat, norm, send_map, recv_map, rank, size,
           W1, b1, W2, b2, W3, b3, W4, b4, W5, b5, W6, b6):
    src = subgraph[0]
    dst = subgraph[1]
    a = _aggregate(_agg1, src, dst, feat * norm)
    h = jax.nn.relu((norm * a) @ W1 + b1)
    for W, b in ((W2, b2), (W3, b3), (W4, b4), (W5, b5)):
        a = _aggregate(_agg3, src, dst, norm * h)
        h = jax.nn.relu((norm * a) @ W + b)
    y = (norm * h) @ W6
    a = _aggregate(_agg1, src, dst, y)
    return jax.nn.relu(norm * a + b6)
